# baseline (speedup 1.0000x reference)
import functools

import jax
import jax.numpy as jnp
from jax import lax
from jax.experimental import pallas as pl
from jax.experimental.pallas import tpu as pltpu
from jax.experimental.pallas import tpu_sc as plsc

B = 8
N = 65536
C = 64
R = 32
V = R * R * R

NC = 2
NS = 16
L = 16
NW = NC * NS

K = 4096
NCHUNK = N // K
CPT = N // NW


def _prep_body(ct_ref, norm_ref, idx_ref):
    x = ct_ref[0]
    mean_pt = jnp.mean(x, axis=0, keepdims=True)
    cent = x - mean_pt
    ssq = jnp.sum(cent * cent, axis=1, keepdims=True)
    denom = jnp.max(jnp.sqrt(ssq)) * 2.0
    norm = cent / denom + 0.5
    scaled = jnp.clip(norm * float(R), 0.0, float(R - 1))
    norm_ref[0] = scaled
    vox = jnp.round(scaled).astype(jnp.int32)
    idx_ref[0] = vox[0:1] * (R * R) + vox[1:2] * R + vox[2:3]


def _prep(ct):
    return pl.pallas_call(
        _prep_body,
        grid=(B,),
        in_specs=[pl.BlockSpec((1, 3, N), lambda b: (b, 0, 0))],
        out_specs=[
            pl.BlockSpec((1, 3, N), lambda b: (b, 0, 0)),
            pl.BlockSpec((1, 1, N), lambda b: (b, 0, 0)),
        ],
        out_shape=[
            jax.ShapeDtypeStruct((B, 3, N), jnp.float32),
            jax.ShapeDtypeStruct((B, 1, N), jnp.int32),
        ],
    )(ct)


_TN = 2048


def _transpose_body(f_ref, o_ref):
    o_ref[0] = f_ref[0].T


def _transpose(features):
    return pl.pallas_call(
        _transpose_body,
        grid=(B, N // _TN),
        in_specs=[pl.BlockSpec((1, _TN, C), lambda b, n: (b, n, 0))],
        out_specs=pl.BlockSpec((1, C, _TN), lambda b, n: (b, 0, n)),
        out_shape=jax.ShapeDtypeStruct((B, C, N), jnp.float32),
    )(features)


def _sc_body(ft_hbm, idx_hbm, sums_hbm, cntp_hbm, g0, g1, gc, f0, f1, ib):
    wid = lax.axis_index("s") * NC + lax.axis_index("c")
    c0 = 2 * wid
    c1 = c0 + 1
    ones = jnp.full((L,), 1.0, dtype=jnp.float32)

    def batch_body(b, _):
        def zero_body(i, _):
            z = jnp.zeros((L,), dtype=jnp.float32)
            g0[pl.ds(i * L, L)] = z
            g1[pl.ds(i * L, L)] = z
            gc[pl.ds(i * L, L)] = z
            return 0

        lax.fori_loop(0, V // L, zero_body, 0)

        def chunk_body(j, _):
            pltpu.sync_copy(idx_hbm.at[b, pl.ds(j * K, K)], ib)
            pltpu.sync_copy(ft_hbm.at[b, c0, pl.ds(j * K, K)], f0)
            pltpu.sync_copy(ft_hbm.at[b, c1, pl.ds(j * K, K)], f1)

            def vec_body(i, _):
                vox = ib[pl.ds(i * L, L)]
                plsc.addupdate_scatter(g0, [vox], f0[pl.ds(i * L, L)])
                plsc.addupdate_scatter(g1, [vox], f1[pl.ds(i * L, L)])
                return 0

            lax.fori_loop(0, K // L, vec_body, 0)

            @pl.when(j == wid // (K // CPT))
            def _():
                off = (wid % (K // CPT)) * CPT

                def cnt_body(i, _):
                    vox = ib[pl.ds(off + i * L, L)]
                    plsc.addupdate_scatter(gc, [vox], ones)
                    return 0

                lax.fori_loop(0, CPT // L, cnt_body, 0)

            return 0

        lax.fori_loop(0, NCHUNK, chunk_body, 0)

        pltpu.sync_copy(g0, sums_hbm.at[b, c0])
        pltpu.sync_copy(g1, sums_hbm.at[b, c1])
        pltpu.sync_copy(gc, cntp_hbm.at[b, wid])
        return 0

    lax.fori_loop(0, B, batch_body, 0)


def _sc_scatter(ftT, idx):
    mesh = plsc.VectorSubcoreMesh(core_axis_name="c", subcore_axis_name="s")
    f = pl.kernel(
        _sc_body,
        compiler_params=pltpu.CompilerParams(needs_layout_passes=False),
        out_type=(
            jax.ShapeDtypeStruct((B, C, V), jnp.float32),
            jax.ShapeDtypeStruct((B, NW, V), jnp.float32),
        ),
        mesh=mesh,
        scratch_types=[
            pltpu.VMEM((V,), jnp.float32),
            pltpu.VMEM((V,), jnp.float32),
            pltpu.VMEM((V,), jnp.float32),
            pltpu.VMEM((K,), jnp.float32),
            pltpu.VMEM((K,), jnp.float32),
            pltpu.VMEM((K,), jnp.int32),
        ],
    )
    return f(ftT, idx)


_VB = 2048


def _finalize_body(s_ref, c_ref, o_ref):
    cnt = jnp.sum(c_ref[0], axis=0, keepdims=True)
    cnt = jnp.clip(cnt, 1.0, None)
    o_ref[0] = s_ref[0] / cnt


def _finalize(sums, cntp):
    return pl.pallas_call(
        _finalize_body,
        grid=(B, V // _VB),
        in_specs=[
            pl.BlockSpec((1, C, _VB), lambda b, v: (b, 0, v)),
            pl.BlockSpec((1, NW, _VB), lambda b, v: (b, 0, v)),
        ],
        out_specs=pl.BlockSpec((1, C, _VB), lambda b, v: (b, 0, v)),
        out_shape=jax.ShapeDtypeStruct((B, C, V), jnp.float32),
    )(sums, cntp)


def kernel(features, coords):
    ct = jnp.transpose(coords, (0, 2, 1))
    norm_t, idx3 = _prep(ct)
    norm_coords = jnp.transpose(norm_t, (0, 2, 1))
    idx = idx3.reshape(B, N)
    ftT = _transpose(features)
    sums, cntp = _sc_scatter(ftT, idx)
    vox = _finalize(sums, cntp)
    return vox.reshape(B, C, R, R, R), norm_coords
